# baseline (speedup 1.0000x reference)
import functools

import jax
import jax.numpy as jnp
from jax import lax
from jax.experimental import pallas as pl
from jax.experimental.pallas import tpu as pltpu
from jax.experimental.pallas import tpu_sc as plsc

BATCH = 16384
DIM = 64
NC = 2
NS = 16
NW = NC * NS
BPW = BATCH // NW
CHUNK = 128
CHUNKS = BPW // CHUNK


def _sc_gather(table, idx2d, name):
    mesh = plsc.VectorSubcoreMesh(core_axis_name="c", subcore_axis_name="s")

    @functools.partial(
        pl.kernel,
        out_type=jax.ShapeDtypeStruct((BATCH, DIM), jnp.float32),
        mesh=mesh,
        compiler_params=pltpu.CompilerParams(use_tc_tiling_on_sc=False),
        scratch_types=[
            pltpu.VMEM((CHUNKS, CHUNK), jnp.int32),
            pltpu.VMEM((BPW, DIM), jnp.float32),
            pltpu.SemaphoreType.DMA,
        ],
        name=name,
    )
    def k(tbl_hbm, idx_hbm, out_hbm, idx_v, rows_v, sem):
        wid = lax.axis_index("s") * NC + lax.axis_index("c")
        base = wid * BPW
        pltpu.sync_copy(idx_hbm.at[pl.ds(wid * CHUNKS, CHUNKS)], idx_v)
        cps = []
        for c in range(CHUNKS):
            cps.append(pltpu.async_copy(
                tbl_hbm.at[idx_v.at[c]], rows_v.at[pl.ds(c * CHUNK, CHUNK)], sem))
        for cp in cps:
            cp.wait()
        pltpu.sync_copy(rows_v, out_hbm.at[pl.ds(base, BPW)])

    return k(table, idx2d)


@jax.jit
def kernel(user, pos, neg, user_table, item_table):
    uidx = jnp.asarray(user, jnp.int32).reshape(BATCH // CHUNK, CHUNK)
    pidx = jnp.asarray(pos, jnp.int32).reshape(BATCH // CHUNK, CHUNK)
    nidx = jnp.asarray(neg, jnp.int32).reshape(BATCH // CHUNK, CHUNK)
    return (_sc_gather(user_table, uidx, "gather_user"),
            _sc_gather(item_table, pidx, "gather_pos"),
            _sc_gather(item_table, nidx, "gather_neg"))


# speedup vs baseline: 1.6702x; 1.6702x over previous
import functools

import jax
import jax.numpy as jnp
from jax import lax
from jax.experimental import pallas as pl
from jax.experimental.pallas import tpu as pltpu
from jax.experimental.pallas import tpu_sc as plsc

BATCH = 16384
DIM = 64
NC = 2
NS = 16
NW = NC * NS
BPW = BATCH // NW
LANES = 16

_MESH = plsc.VectorSubcoreMesh(core_axis_name="c", subcore_axis_name="s")
_OUT = jax.ShapeDtypeStruct((BATCH, DIM), jnp.float32)


def _fire_rows(tbl, idx_hbm, gbuf, sem, base):
    def grp(g, carry):
        gb = g * LANES
        v = idx_hbm[pl.ds(gb, LANES)]
        for j in range(LANES):
            pltpu.async_copy(
                tbl.at[pl.ds(v[j], 1)], gbuf.at[pl.ds(gb + j, 1)], sem)
        return carry

    lax.fori_loop(0, BPW // LANES, grp, 0)


def _drain_and_store(tbl, gbuf, sem, out_hbm, base):
    pltpu.make_async_copy(tbl.at[pl.ds(0, BPW)], gbuf, sem).wait()
    pltpu.sync_copy(gbuf, out_hbm.at[pl.ds(base, BPW)])


def _sc_gather_user(user_table, uidx):
    @functools.partial(
        pl.kernel,
        out_type=_OUT,
        mesh=_MESH,
        scratch_types=[
            pltpu.VMEM((BPW,), jnp.int32),
            pltpu.VMEM((BPW, DIM), jnp.float32),
            pltpu.SemaphoreType.DMA,
        ],
        name="gather_user",
    )
    def k(tbl, ui_hbm, out_u, idxbuf, gbuf, sem):
        wid = lax.axis_index("s") * NC + lax.axis_index("c")
        base = wid * BPW
        pltpu.sync_copy(ui_hbm.at[pl.ds(base, BPW)], idxbuf)
        _fire_rows(tbl, idxbuf, gbuf, sem, base)
        _drain_and_store(tbl, gbuf, sem, out_u, base)

    return k(user_table, uidx)


def _sc_gather_items(item_table, pidx, nidx):
    @functools.partial(
        pl.kernel,
        out_type=(_OUT, _OUT),
        mesh=_MESH,
        scratch_types=[
            pltpu.VMEM((BPW,), jnp.int32),
            pltpu.VMEM((BPW, DIM), jnp.float32),
            pltpu.SemaphoreType.DMA,
        ],
        name="gather_items",
    )
    def k(tbl, pi_hbm, ni_hbm, out_p, out_n, idxbuf, gbuf, sem):
        wid = lax.axis_index("s") * NC + lax.axis_index("c")
        base = wid * BPW
        pltpu.sync_copy(pi_hbm.at[pl.ds(base, BPW)], idxbuf)
        _fire_rows(tbl, idxbuf, gbuf, sem, base)
        _drain_and_store(tbl, gbuf, sem, out_p, base)
        pltpu.sync_copy(ni_hbm.at[pl.ds(base, BPW)], idxbuf)
        _fire_rows(tbl, idxbuf, gbuf, sem, base)
        _drain_and_store(tbl, gbuf, sem, out_n, base)

    return k(item_table, pidx, nidx)


@jax.jit
def kernel(user, pos, neg, user_table, item_table):
    uidx = jnp.asarray(user, jnp.int32)
    pidx = jnp.asarray(pos, jnp.int32)
    nidx = jnp.asarray(neg, jnp.int32)
    out_u = _sc_gather_user(user_table, uidx)
    out_p, out_n = _sc_gather_items(item_table, pidx, nidx)
    return (out_u, out_p, out_n)
